# baseline (speedup 1.0000x reference)
import functools
import math

import jax
import jax.numpy as jnp
from jax import lax
from jax.experimental import pallas as pl
from jax.experimental.pallas import tpu as pltpu
from jax.experimental.pallas import tpu_sc as plsc

_SIZE = 32000
_PAD = 0
_SMOOTH = 0.1
_CONF = 1.0 - _SMOOTH
_FILL = _SMOOTH / (_SIZE - 2)
_C0 = _CONF * math.log(_CONF) + _SMOOTH * math.log(_FILL)

_ROWS = 128

_NC, _NS, _L = 2, 16, 16
_NW = _NC * _NS
_N = 2048
_BPW = _N // _NW
_CPR = _SIZE // 128
_TPR = 8 * _CPR


def _tc_body(t_ref, x_ref, out_ref):
    j = pl.program_id(0)
    x = x_ref[...]
    r, size = x.shape
    t = t_ref[:, 0]

    p = jnp.zeros((r, 128), jnp.float32)
    for k in range(size // 128):
        p = p + x[:, k * 128:(k + 1) * 128]

    ones = jnp.ones((128, 1), jnp.float32)
    rs = jax.lax.dot(p, ones, preferred_element_type=jnp.float32)[:, 0]
    validf = (t != _PAD).astype(jnp.float32)
    partial = jnp.sum(validf * rs)

    @pl.when(j == 0)
    def _():
        out_ref[...] = partial.reshape(1, 1)

    @pl.when(j > 0)
    def _():
        out_ref[...] += partial.reshape(1, 1)


_sc_mesh = plsc.VectorSubcoreMesh(
    core_axis_name="c", subcore_axis_name="s",
    num_cores=_NC, num_subcores=_NS)


@functools.partial(
    pl.kernel,
    out_type=(jax.ShapeDtypeStruct((_NW, _L), jnp.float32),
              jax.ShapeDtypeStruct((_NW, _L), jnp.float32),
              jax.ShapeDtypeStruct((_NW, _L), jnp.float32)),
    mesh=_sc_mesh,
    compiler_params=pltpu.CompilerParams(needs_layout_passes=False),
    scratch_types=[
        pltpu.VMEM((_BPW,), jnp.int32),
        pltpu.VMEM((_BPW,), jnp.int32),
        pltpu.VMEM((_BPW,), jnp.int32),
        pltpu.VMEM((_BPW, 128), jnp.float32),
        pltpu.VMEM((_BPW, 128), jnp.float32),
        pltpu.VMEM((_L,), jnp.float32),
        pltpu.VMEM((_L,), jnp.float32),
        pltpu.VMEM((_L,), jnp.float32),
        pltpu.SemaphoreType.DMA,
    ],
)
def _sc_gather(xr_hbm, tgt_hbm, out_t, out_0, out_n,
               tgt_v, idx_v, idx0_v, gt_v, g0_v, at_v, a0_v, an_v, sem):
    wid = lax.axis_index("s") * _NC + lax.axis_index("c")
    base = wid * _BPW
    pltpu.sync_copy(tgt_hbm.at[pl.ds(base, _BPW)], tgt_v)
    for g in range(_BPW // _L):
        tv = tgt_v[pl.ds(g * _L, _L)]
        rowv = base + g * _L + lax.iota(jnp.int32, _L)
        idx_v[pl.ds(g * _L, _L)] = rowv * _CPR + (tv >> 7)
        idx0_v[pl.ds(g * _L, _L)] = rowv * _CPR
    pltpu.async_copy(xr_hbm.at[idx_v], gt_v, sem).wait()
    pltpu.async_copy(xr_hbm.at[idx0_v], g0_v, sem).wait()
    acc_t = jnp.zeros((_L,), jnp.float32)
    acc_0 = jnp.zeros((_L,), jnp.float32)
    acc_n = jnp.zeros((_L,), jnp.float32)
    zero = jnp.zeros((_L,), jnp.int32)
    for g in range(_BPW // _L):
        sl = pl.ds(g * _L, _L)
        tv = tgt_v[sl]
        lrow = g * _L + lax.iota(jnp.int32, _L)
        xt = plsc.load_gather(gt_v, [lrow, tv & 127])
        x0 = plsc.load_gather(g0_v, [lrow, zero])
        valid = tv != _PAD
        acc_t += jnp.where(valid, xt, 0.0)
        acc_0 += jnp.where(valid, x0, 0.0)
        acc_n += jnp.where(valid, 1.0, 0.0)
    at_v[...] = acc_t
    a0_v[...] = acc_0
    an_v[...] = acc_n
    pltpu.sync_copy(at_v, out_t.at[wid])
    pltpu.sync_copy(a0_v, out_0.at[wid])
    pltpu.sync_copy(an_v, out_n.at[wid])


@jax.jit
def kernel(x, target):
    n, size = x.shape
    t2 = target.reshape(n, 1)
    grid = n // _ROWS
    s_dense = pl.pallas_call(
        _tc_body,
        grid=(grid,),
        in_specs=[
            pl.BlockSpec((_ROWS, 1), lambda j: (j, 0)),
            pl.BlockSpec((_ROWS, size), lambda j: (j, 0)),
        ],
        out_specs=pl.BlockSpec((1, 1), lambda j: (0, 0)),
        out_shape=jax.ShapeDtypeStruct((1, 1), jnp.float32),
    )(t2, x)[0, 0]

    xr = x.reshape(n * size // 128, 128)
    parts_t, parts_0, parts_n = _sc_gather(xr, target)
    s_t = jnp.sum(parts_t)
    s_0 = jnp.sum(parts_0)
    nv = jnp.sum(parts_n)
    return nv * _C0 - (_CONF - _FILL) * s_t - _FILL * (s_dense - s_0)


# speedup vs baseline: 2.4516x; 2.4516x over previous
import functools
import math

import jax
import jax.numpy as jnp
from jax import lax
from jax.experimental import pallas as pl
from jax.experimental.pallas import tpu as pltpu
from jax.experimental.pallas import tpu_sc as plsc

_SIZE = 32000
_PAD = 0
_SMOOTH = 0.1
_CONF = 1.0 - _SMOOTH
_FILL = _SMOOTH / (_SIZE - 2)
_C0 = _CONF * math.log(_CONF) + _SMOOTH * math.log(_FILL)

_ROWS = 128

_NC, _NS, _L = 2, 16, 16
_NW = _NC * _NS
_N = 2048
_BPW = _N // _NW


def _tc_body(t_ref, x_ref, out_ref, pay_ref):
    j = pl.program_id(0)
    x = x_ref[...]
    r, size = x.shape
    t = t_ref[:, 0]
    tchunk = (t >> 7)[:, None]

    p = jnp.zeros((r, 128), jnp.float32)
    pay = jnp.zeros((r, 128), jnp.float32)
    for k in range(size // 128):
        xs = x[:, k * 128:(k + 1) * 128]
        p = p + xs
        pay = jnp.where(tchunk == k, xs, pay)

    ones = jnp.ones((128, 1), jnp.float32)
    rs = jax.lax.dot(p, ones, preferred_element_type=jnp.float32)[:, 0]
    validf = (t != _PAD).astype(jnp.float32)
    partial = jnp.sum(validf * (rs - x[:, 0]))

    pay_ref[...] = pay

    @pl.when(j == 0)
    def _():
        out_ref[...] = partial.reshape(1, 1)

    @pl.when(j > 0)
    def _():
        out_ref[...] += partial.reshape(1, 1)


_sc_mesh = plsc.VectorSubcoreMesh(
    core_axis_name="c", subcore_axis_name="s",
    num_cores=_NC, num_subcores=_NS)


@functools.partial(
    pl.kernel,
    out_type=(jax.ShapeDtypeStruct((_NW, _L), jnp.float32),
              jax.ShapeDtypeStruct((_NW, _L), jnp.float32)),
    mesh=_sc_mesh,
    compiler_params=pltpu.CompilerParams(needs_layout_passes=False),
    scratch_types=[
        pltpu.VMEM((_BPW,), jnp.int32),
        pltpu.VMEM((_BPW, 128), jnp.float32),
        pltpu.VMEM((_L,), jnp.float32),
        pltpu.VMEM((_L,), jnp.float32),
    ],
)
def _sc_pick(pay_hbm, tgt_hbm, out_t, out_n, tgt_v, pay_v, at_v, an_v):
    wid = lax.axis_index("s") * _NC + lax.axis_index("c")
    base = wid * _BPW
    pltpu.sync_copy(tgt_hbm.at[pl.ds(base, _BPW)], tgt_v)
    pltpu.sync_copy(pay_hbm.at[pl.ds(base, _BPW)], pay_v)
    acc_t = jnp.zeros((_L,), jnp.float32)
    acc_n = jnp.zeros((_L,), jnp.float32)
    for g in range(_BPW // _L):
        tv = tgt_v[pl.ds(g * _L, _L)]
        lrow = g * _L + lax.iota(jnp.int32, _L)
        xt = plsc.load_gather(pay_v, [lrow, tv & 127])
        valid = tv != _PAD
        acc_t += jnp.where(valid, xt, 0.0)
        acc_n += jnp.where(valid, 1.0, 0.0)
    at_v[...] = acc_t
    an_v[...] = acc_n
    pltpu.sync_copy(at_v, out_t.at[wid])
    pltpu.sync_copy(an_v, out_n.at[wid])


@jax.jit
def kernel(x, target):
    n, size = x.shape
    t2 = target.reshape(n, 1)
    grid = n // _ROWS
    s_adj, payload = pl.pallas_call(
        _tc_body,
        grid=(grid,),
        in_specs=[
            pl.BlockSpec((_ROWS, 1), lambda j: (j, 0)),
            pl.BlockSpec((_ROWS, size), lambda j: (j, 0)),
        ],
        out_specs=(pl.BlockSpec((1, 1), lambda j: (0, 0)),
                   pl.BlockSpec((_ROWS, 128), lambda j: (j, 0))),
        out_shape=(jax.ShapeDtypeStruct((1, 1), jnp.float32),
                   jax.ShapeDtypeStruct((n, 128), jnp.float32)),
    )(t2, x)

    parts_t, parts_n = _sc_pick(payload, target)
    s_t = jnp.sum(parts_t)
    nv = jnp.sum(parts_n)
    return nv * _C0 - (_CONF - _FILL) * s_t - _FILL * s_adj[0, 0]


# speedup vs baseline: 2.7741x; 1.1316x over previous
import functools
import math

import jax
import jax.numpy as jnp
from jax import lax
from jax.experimental import pallas as pl
from jax.experimental.pallas import tpu as pltpu
from jax.experimental.pallas import tpu_sc as plsc

_SIZE = 32000
_PAD = 0
_SMOOTH = 0.1
_CONF = 1.0 - _SMOOTH
_FILL = _SMOOTH / (_SIZE - 2)
_C0 = _CONF * math.log(_CONF) + _SMOOTH * math.log(_FILL)
_K = _CONF / _FILL

_ROWS = 128

_NC, _NS, _L = 2, 16, 16
_NW = _NC * _NS
_N = 2048
_BPW = _N // _NW


def _tc_body(t_ref, x_ref, out_ref):
    j = pl.program_id(0)
    x = x_ref[...]
    r, size = x.shape
    t = t_ref[:, 0]

    p = jnp.zeros((r, 128), jnp.float32)
    for k in range(size // 128):
        xs = x[:, k * 128:(k + 1) * 128]
        cid = k * 128 + jax.lax.broadcasted_iota(jnp.int32, (r, 128), 1)
        z = jnp.where(cid == t[:, None], _K * xs, xs)
        if k == 0:
            z = jnp.where(cid == 0, 0.0, z)
        p = p + z

    ones = jnp.ones((128, 1), jnp.float32)
    rowz = jax.lax.dot(p, ones, preferred_element_type=jnp.float32)[:, 0]
    validf = (t != _PAD).astype(jnp.float32)
    partial = jnp.sum(validf * rowz)

    @pl.when(j == 0)
    def _():
        out_ref[...] = partial.reshape(1, 1)

    @pl.when(j > 0)
    def _():
        out_ref[...] += partial.reshape(1, 1)


_sc_mesh = plsc.VectorSubcoreMesh(
    core_axis_name="c", subcore_axis_name="s",
    num_cores=_NC, num_subcores=_NS)


@functools.partial(
    pl.kernel,
    out_type=jax.ShapeDtypeStruct((_NW, _L), jnp.float32),
    mesh=_sc_mesh,
    scratch_types=[
        pltpu.VMEM((_BPW,), jnp.int32),
        pltpu.VMEM((_L,), jnp.float32),
    ],
)
def _sc_count(tgt_hbm, out_n, tgt_v, an_v):
    wid = lax.axis_index("s") * _NC + lax.axis_index("c")
    base = wid * _BPW
    pltpu.sync_copy(tgt_hbm.at[pl.ds(base, _BPW)], tgt_v)
    acc_n = jnp.zeros((_L,), jnp.float32)
    for g in range(_BPW // _L):
        tv = tgt_v[pl.ds(g * _L, _L)]
        acc_n += jnp.where(tv != _PAD, 1.0, 0.0)
    an_v[...] = acc_n
    pltpu.sync_copy(an_v, out_n.at[wid])


@jax.jit
def kernel(x, target):
    n, size = x.shape
    t2 = target.reshape(n, 1)
    grid = n // _ROWS
    s = pl.pallas_call(
        _tc_body,
        grid=(grid,),
        in_specs=[
            pl.BlockSpec((_ROWS, 1), lambda j: (j, 0)),
            pl.BlockSpec((_ROWS, size), lambda j: (j, 0)),
        ],
        out_specs=pl.BlockSpec((1, 1), lambda j: (0, 0)),
        out_shape=jax.ShapeDtypeStruct((1, 1), jnp.float32),
    )(t2, x)[0, 0]

    nv = jnp.sum(_sc_count(target))
    return nv * _C0 - _FILL * s


# speedup vs baseline: 3.3562x; 1.2098x over previous
import math

import jax
import jax.numpy as jnp
from jax.experimental import pallas as pl
from jax.experimental.pallas import tpu as pltpu

_SIZE = 32000
_PAD = 0
_SMOOTH = 0.1
_CONF = 1.0 - _SMOOTH
_FILL = _SMOOTH / (_SIZE - 2)
_C0 = _CONF * math.log(_CONF) + _SMOOTH * math.log(_FILL)
_K = _CONF / _FILL

_ROWS = 64


def _body(t_ref, x_ref, out_ref):
    j = pl.program_id(0)
    x = x_ref[...]
    r, size = x.shape
    t = t_ref[:, 0]

    p = jnp.zeros((r, 128), jnp.float32)
    for k in range(size // 128):
        xs = x[:, k * 128:(k + 1) * 128]
        cid = k * 128 + jax.lax.broadcasted_iota(jnp.int32, (r, 128), 1)
        z = jnp.where(cid == t[:, None], _K * xs, xs)
        if k == 0:
            z = jnp.where(cid == 0, 0.0, z)
        p = p + z

    ones = jnp.ones((128, 1), jnp.float32)
    rowz = jax.lax.dot(p, ones, preferred_element_type=jnp.float32)[:, 0]
    validf = (t != _PAD).astype(jnp.float32)
    partial = jnp.sum(validf) * _C0 - _FILL * jnp.sum(validf * rowz)

    @pl.when(j == 0)
    def _():
        out_ref[...] = partial.reshape(1, 1)

    @pl.when(j > 0)
    def _():
        out_ref[...] += partial.reshape(1, 1)


@jax.jit
def kernel(x, target):
    n, size = x.shape
    t2 = target.reshape(n, 1)
    grid = n // _ROWS
    out = pl.pallas_call(
        _body,
        grid=(grid,),
        in_specs=[
            pl.BlockSpec((_ROWS, 1), lambda j: (j, 0)),
            pl.BlockSpec((_ROWS, size), lambda j: (j, 0)),
        ],
        out_specs=pl.BlockSpec((1, 1), lambda j: (0, 0)),
        out_shape=jax.ShapeDtypeStruct((1, 1), jnp.float32),
    )(t2, x)
    return out[0, 0]
